# Initial kernel scaffold; baseline (speedup 1.0000x reference)
import jax
import jax.numpy as jnp
from jax.experimental import pallas as pl


def kernel(x, edge_index, batch, W1, b1, W2, b2, fc1_W, fc1_b, fc2_W, fc2_b):
    raise NotImplementedError("write your pallas kernel here")


# speedup vs baseline: 8.2597x; 8.2597x over previous
import functools

import jax
import jax.numpy as jnp
from jax import lax
from jax.experimental import pallas as pl
from jax.experimental.pallas import tpu as pltpu
from jax.experimental.pallas import tpu_sc as plsc

_G = 128
_NC = 2
_NS = 16
_CH = 128
_BM = 1000


def _sc_degree(dst_pad, ones16, zeros16, np_rows):
    e_pad = dst_pad.shape[0]
    per_w = e_pad // (_NC * _NS)
    n_ch = per_w // _CH
    stripe = np_rows // _NS
    mesh = plsc.VectorSubcoreMesh(core_axis_name="c", subcore_axis_name="s")

    @functools.partial(
        pl.kernel,
        mesh=mesh,
        out_type=jax.ShapeDtypeStruct((_NC * np_rows, 16), jnp.float32),
        scratch_types=[
            pltpu.VMEM((_CH,), jnp.int32),
            pltpu.VMEM((_CH, 16), jnp.float32),
            pltpu.VMEM_SHARED((np_rows, 16), jnp.float32),
        ],
    )
    def k(dst_hbm, ones_hbm, z_hbm, out_hbm, didx, ones, acc):
        cid = lax.axis_index("c")
        sid = lax.axis_index("s")
        base = (cid * _NS + sid) * per_w

        pltpu.sync_copy(ones_hbm, ones)
        pltpu.sync_copy(z_hbm.at[pl.ds(sid * stripe, stripe)],
                        acc.at[pl.ds(sid * stripe, stripe)])
        plsc.subcore_barrier()

        @pl.loop(0, n_ch)
        def _(i):
            pltpu.sync_copy(dst_hbm.at[pl.ds(base + i * _CH, _CH)], didx)
            pltpu.sync_copy(ones, acc.at[didx], add=True)

        plsc.subcore_barrier()
        pltpu.sync_copy(acc.at[pl.ds(sid * stripe, stripe)],
                        out_hbm.at[pl.ds(cid * np_rows + sid * stripe, stripe)])

    return k(dst_pad, ones16, zeros16)


def _sc_aggregate(h_pad, src_pad, dst_pad, zeros, np_rows):
    e_pad = src_pad.shape[0]
    d = h_pad.shape[1]
    per_w = e_pad // (_NC * _NS)
    n_ch = per_w // _CH
    stripe = np_rows // _NS
    mesh = plsc.VectorSubcoreMesh(core_axis_name="c", subcore_axis_name="s")

    @functools.partial(
        pl.kernel,
        mesh=mesh,
        out_type=jax.ShapeDtypeStruct((_NC * np_rows, d), jnp.float32),
        scratch_types=[
            pltpu.VMEM((_CH,), jnp.int32),
            pltpu.VMEM((_CH,), jnp.int32),
            pltpu.VMEM((_CH, d), jnp.float32),
            pltpu.VMEM_SHARED((np_rows, d), jnp.float32),
            pltpu.SemaphoreType.DMA,
        ],
    )
    def k(h_hbm, s_hbm, d_hbm, z_hbm, out_hbm, sidx, didx, rows, acc, sem):
        cid = lax.axis_index("c")
        sid = lax.axis_index("s")
        base = (cid * _NS + sid) * per_w

        pltpu.sync_copy(z_hbm.at[pl.ds(sid * stripe, stripe)],
                        acc.at[pl.ds(sid * stripe, stripe)])
        plsc.subcore_barrier()

        @pl.loop(0, n_ch)
        def _(i):
            b = base + i * _CH
            pltpu.sync_copy(s_hbm.at[pl.ds(b, _CH)], sidx)
            pltpu.sync_copy(d_hbm.at[pl.ds(b, _CH)], didx)
            pltpu.async_copy(h_hbm.at[sidx], rows, sem).wait()
            pltpu.sync_copy(rows, acc.at[didx], add=True)

        plsc.subcore_barrier()
        pltpu.sync_copy(acc.at[pl.ds(sid * stripe, stripe)],
                        out_hbm.at[pl.ds(cid * np_rows + sid * stripe, stripe)])

    return k(h_pad, src_pad, dst_pad, zeros)


def _tc_matmul(x, w):
    n, d = x.shape
    h = w.shape[1]

    def body(x_ref, w_ref, o_ref):
        o_ref[...] = jnp.dot(x_ref[...], w_ref[...],
                             preferred_element_type=jnp.float32)

    return pl.pallas_call(
        body,
        grid=(n // _BM,),
        in_specs=[
            pl.BlockSpec((_BM, d), lambda i: (i, 0)),
            pl.BlockSpec((d, h), lambda i: (0, 0)),
        ],
        out_specs=pl.BlockSpec((_BM, h), lambda i: (i, 0)),
        out_shape=jax.ShapeDtypeStruct((n, h), jnp.float32),
    )(x, w)


def _tc_scale(deg_p, h1, n):
    d = h1.shape[1]

    def body(deg_ref, h_ref, hs_ref, dinv_ref):
        deg = deg_ref[0, :, 0:1] + deg_ref[1, :, 0:1] + 1.0
        dinv = lax.rsqrt(deg)
        dinvb = jnp.broadcast_to(dinv, (_BM, d))
        hs_ref[...] = h_ref[...] * dinvb
        dinv_ref[...] = dinvb

    return pl.pallas_call(
        body,
        grid=(n // _BM,),
        in_specs=[
            pl.BlockSpec((2, _BM, 16), lambda i: (0, i, 0)),
            pl.BlockSpec((_BM, d), lambda i: (i, 0)),
        ],
        out_specs=[
            pl.BlockSpec((_BM, d), lambda i: (i, 0)),
            pl.BlockSpec((_BM, d), lambda i: (i, 0)),
        ],
        out_shape=[
            jax.ShapeDtypeStruct((n, d), jnp.float32),
            jax.ShapeDtypeStruct((n, d), jnp.float32),
        ],
    )(deg_p, h1)


def _tc_mid(agg1, h1s, dinvb, b1, w2, n):
    d = h1s.shape[1]

    def body(a_ref, hs_ref, di_ref, b_ref, w_ref, o_ref):
        t = (a_ref[0] + a_ref[1] + hs_ref[...]) * di_ref[...] + b_ref[...]
        t = jnp.maximum(t, 0.0)
        h2 = jnp.dot(t, w_ref[...], preferred_element_type=jnp.float32)
        o_ref[...] = h2 * di_ref[...]

    return pl.pallas_call(
        body,
        grid=(n // _BM,),
        in_specs=[
            pl.BlockSpec((2, _BM, d), lambda i: (0, i, 0)),
            pl.BlockSpec((_BM, d), lambda i: (i, 0)),
            pl.BlockSpec((_BM, d), lambda i: (i, 0)),
            pl.BlockSpec((1, d), lambda i: (0, 0)),
            pl.BlockSpec((d, d), lambda i: (0, 0)),
        ],
        out_specs=pl.BlockSpec((_BM, d), lambda i: (i, 0)),
        out_shape=jax.ShapeDtypeStruct((n, d), jnp.float32),
    )(agg1, h1s, dinvb, b1, w2)


def _tc_tail(agg2, h2s, dinvb, b2, batch_r, fc1_w, fc1_b, fc2_w, fc2_b, n):
    d = h2s.shape[1]
    c = fc2_w.shape[1]
    nblk = n // _BM

    def body(a_ref, hs_ref, di_ref, b_ref, bat_ref, w1_ref, bb1_ref,
             w2_ref, bb2_ref, o_ref, pooled):
        i = pl.program_id(0)
        t = (a_ref[0] + a_ref[1] + hs_ref[...]) * di_ref[...] + b_ref[...]
        t = jnp.maximum(t, 0.0)
        bvec = bat_ref[0, 0, :]
        oh = (bvec[:, None] == lax.broadcasted_iota(jnp.int32, (_BM, _G), 1))
        oh = oh.astype(jnp.float32)
        contrib = lax.dot_general(oh, t, (((0,), (0,)), ((), ())),
                                  preferred_element_type=jnp.float32)

        @pl.when(i == 0)
        def _():
            pooled[...] = contrib

        @pl.when(i > 0)
        def _():
            pooled[...] += contrib

        @pl.when(i == nblk - 1)
        def _():
            p = pooled[...]
            f = jnp.dot(p, w1_ref[...], preferred_element_type=jnp.float32)
            f = jnp.maximum(f + bb1_ref[...], 0.0)
            logits = jnp.dot(f, w2_ref[...],
                             preferred_element_type=jnp.float32) + bb2_ref[...]
            m = jnp.max(logits, axis=-1, keepdims=True)
            lse = m + jnp.log(jnp.sum(jnp.exp(logits - m), axis=-1,
                                      keepdims=True))
            o_ref[...] = logits - lse

    return pl.pallas_call(
        body,
        grid=(nblk,),
        in_specs=[
            pl.BlockSpec((2, _BM, d), lambda i: (0, i, 0)),
            pl.BlockSpec((_BM, d), lambda i: (i, 0)),
            pl.BlockSpec((_BM, d), lambda i: (i, 0)),
            pl.BlockSpec((1, d), lambda i: (0, 0)),
            pl.BlockSpec((1, 1, _BM), lambda i: (i, 0, 0)),
            pl.BlockSpec((d, d), lambda i: (0, 0)),
            pl.BlockSpec((1, d), lambda i: (0, 0)),
            pl.BlockSpec((d, c), lambda i: (0, 0)),
            pl.BlockSpec((1, c), lambda i: (0, 0)),
        ],
        out_specs=pl.BlockSpec((_G, c), lambda i: (0, 0)),
        out_shape=jax.ShapeDtypeStruct((_G, c), jnp.float32),
        scratch_shapes=[pltpu.VMEM((_G, d), jnp.float32)],
    )(agg2, h2s, dinvb, b2, batch_r, fc1_w, fc1_b, fc2_w, fc2_b)


def kernel(x, edge_index, batch, W1, b1, W2, b2, fc1_W, fc1_b, fc2_W, fc2_b):
    n, d = x.shape
    e = edge_index.shape[1]
    nw = _NC * _NS

    n_ch = -(-e // (nw * _CH))
    if n_ch % 2:
        n_ch += 1
    e_pad = n_ch * _CH * nw
    np_rows = ((n + 1 + 127) // 128) * 128

    src_p = jnp.concatenate(
        [edge_index[0], jnp.full((e_pad - e,), n, jnp.int32)])
    dst_p = jnp.concatenate(
        [edge_index[1], jnp.full((e_pad - e,), n, jnp.int32)])
    ones16 = jnp.ones((_CH, 16), jnp.float32)
    zeros16 = jnp.zeros((np_rows, 16), jnp.float32)
    zeros_d = jnp.zeros((np_rows, d), jnp.float32)
    pad_rows = jnp.zeros((np_rows - n, d), jnp.float32)

    deg_p = _sc_degree(dst_p, ones16, zeros16, np_rows).reshape(_NC, np_rows, 16)
    h1 = _tc_matmul(x, W1)
    h1s, dinvb = _tc_scale(deg_p, h1, n)

    agg1 = _sc_aggregate(jnp.concatenate([h1s, pad_rows]), src_p, dst_p,
                         zeros_d, np_rows).reshape(_NC, np_rows, d)
    h2s = _tc_mid(agg1, h1s, dinvb, b1.reshape(1, -1), W2, n)

    agg2 = _sc_aggregate(jnp.concatenate([h2s, pad_rows]), src_p, dst_p,
                         zeros_d, np_rows).reshape(_NC, np_rows, d)
    nblk = n // _BM
    out = _tc_tail(agg2, h2s, dinvb, b2.reshape(1, -1),
                   batch.reshape(nblk, 1, _BM), fc1_W, fc1_b.reshape(1, -1),
                   fc2_W, fc2_b.reshape(1, -1), n)
    return out


# speedup vs baseline: 9.0810x; 1.0994x over previous
import functools

import jax
import jax.numpy as jnp
from jax import lax
from jax.experimental import pallas as pl
from jax.experimental.pallas import tpu as pltpu
from jax.experimental.pallas import tpu_sc as plsc

_G = 128
_NC = 2
_NS = 16
_CH = 128
_BM = 1000


def _sc_degree(dst_flat, ones16, zeros16, np_rows):
    n_ch = dst_flat.shape[0] // (_NC * _NS * _CH)
    stripe = np_rows // _NS
    mesh = plsc.VectorSubcoreMesh(core_axis_name="c", subcore_axis_name="s")

    @functools.partial(
        pl.kernel,
        mesh=mesh,
        out_type=jax.ShapeDtypeStruct((_NC * np_rows, 16), jnp.float32),
        scratch_types=[
            pltpu.VMEM((_CH,), jnp.int32),
            pltpu.VMEM((_CH, 16), jnp.float32),
            pltpu.VMEM_SHARED((np_rows, 16), jnp.float32),
        ],
    )
    def k(dst_hbm, ones_hbm, z_hbm, out_hbm, didx, ones, acc):
        cid = lax.axis_index("c")
        sid = lax.axis_index("s")
        wid = cid * _NS + sid

        pltpu.sync_copy(ones_hbm, ones)
        pltpu.sync_copy(z_hbm.at[pl.ds(sid * stripe, stripe)],
                        acc.at[pl.ds(sid * stripe, stripe)])
        plsc.subcore_barrier()

        @pl.loop(0, n_ch)
        def _(i):
            pltpu.sync_copy(
                dst_hbm.at[pl.ds((wid * n_ch + i) * _CH, _CH)], didx)
            pltpu.sync_copy(ones, acc.at[didx], add=True)

        plsc.subcore_barrier()
        pltpu.sync_copy(acc.at[pl.ds(sid * stripe, stripe)],
                        out_hbm.at[pl.ds(cid * np_rows + sid * stripe, stripe)])

    return k(dst_flat, ones16, zeros16)


def _sc_aggregate(h_pad, src2, dst2, zeros, np_rows):
    d = h_pad.shape[1]
    n_ch = src2.shape[0] // (_NC * _NS)
    n_ph = n_ch // 2
    stripe = np_rows // _NS
    mesh = plsc.VectorSubcoreMesh(core_axis_name="c", subcore_axis_name="s")

    @functools.partial(
        pl.kernel,
        mesh=mesh,
        out_type=jax.ShapeDtypeStruct((_NC * np_rows, d), jnp.float32),
        scratch_types=[
            pltpu.VMEM((n_ph, _CH), jnp.int32),
            pltpu.VMEM((n_ph, _CH), jnp.int32),
            pltpu.VMEM((_CH, d), jnp.float32),
            pltpu.VMEM((_CH, d), jnp.float32),
            pltpu.VMEM_SHARED((np_rows, d), jnp.float32),
            pltpu.SemaphoreType.DMA,
            pltpu.SemaphoreType.DMA,
        ],
    )
    def k(h_hbm, s_hbm, d_hbm, z_hbm, out_hbm, sidx, didx, rows0, rows1,
          acc, sem0, sem1):
        cid = lax.axis_index("c")
        sid = lax.axis_index("s")
        wid = cid * _NS + sid

        pltpu.sync_copy(z_hbm.at[pl.ds(sid * stripe, stripe)],
                        acc.at[pl.ds(sid * stripe, stripe)])
        plsc.subcore_barrier()

        @pl.loop(0, 2)
        def _(p):
            pltpu.sync_copy(s_hbm.at[pl.ds(wid * n_ch + p * n_ph, n_ph)], sidx)
            pltpu.sync_copy(d_hbm.at[pl.ds(wid * n_ch + p * n_ph, n_ph)], didx)
            pltpu.async_copy(h_hbm.at[sidx.at[0]], rows0, sem0)

            @pl.loop(0, n_ph // 2)
            def _(j):
                i0 = 2 * j
                pltpu.make_async_copy(h_hbm.at[sidx.at[i0]], rows0,
                                      sem0).wait()
                pltpu.async_copy(h_hbm.at[sidx.at[i0 + 1]], rows1, sem1)
                pltpu.sync_copy(rows0, acc.at[didx.at[i0]], add=True)
                pltpu.make_async_copy(h_hbm.at[sidx.at[i0 + 1]], rows1,
                                      sem1).wait()

                @pl.when(j < n_ph // 2 - 1)
                def _():
                    pltpu.async_copy(h_hbm.at[sidx.at[i0 + 2]], rows0, sem0)

                pltpu.sync_copy(rows1, acc.at[didx.at[i0 + 1]], add=True)

        plsc.subcore_barrier()
        pltpu.sync_copy(acc.at[pl.ds(sid * stripe, stripe)],
                        out_hbm.at[pl.ds(cid * np_rows + sid * stripe, stripe)])

    return k(h_pad, src2, dst2, zeros)


def _tc_matmul(x, w):
    n, d = x.shape
    h = w.shape[1]

    def body(x_ref, w_ref, o_ref):
        o_ref[...] = jnp.dot(x_ref[...], w_ref[...],
                             preferred_element_type=jnp.float32)

    return pl.pallas_call(
        body,
        grid=(n // _BM,),
        in_specs=[
            pl.BlockSpec((_BM, d), lambda i: (i, 0)),
            pl.BlockSpec((d, h), lambda i: (0, 0)),
        ],
        out_specs=pl.BlockSpec((_BM, h), lambda i: (i, 0)),
        out_shape=jax.ShapeDtypeStruct((n, h), jnp.float32),
    )(x, w)


def _tc_scale(deg_p, h1, n):
    d = h1.shape[1]

    def body(deg_ref, h_ref, hs_ref, dinv_ref):
        deg = deg_ref[0, :, 0:1] + deg_ref[1, :, 0:1] + 1.0
        dinv = lax.rsqrt(deg)
        dinvb = jnp.broadcast_to(dinv, (_BM, d))
        hs_ref[...] = h_ref[...] * dinvb
        dinv_ref[...] = dinvb

    return pl.pallas_call(
        body,
        grid=(n // _BM,),
        in_specs=[
            pl.BlockSpec((2, _BM, 16), lambda i: (0, i, 0)),
            pl.BlockSpec((_BM, d), lambda i: (i, 0)),
        ],
        out_specs=[
            pl.BlockSpec((_BM, d), lambda i: (i, 0)),
            pl.BlockSpec((_BM, d), lambda i: (i, 0)),
        ],
        out_shape=[
            jax.ShapeDtypeStruct((n, d), jnp.float32),
            jax.ShapeDtypeStruct((n, d), jnp.float32),
        ],
    )(deg_p, h1)


def _tc_mid(agg1, h1s, dinvb, b1, w2, n):
    d = h1s.shape[1]

    def body(a_ref, hs_ref, di_ref, b_ref, w_ref, o_ref):
        t = (a_ref[0] + a_ref[1] + hs_ref[...]) * di_ref[...] + b_ref[...]
        t = jnp.maximum(t, 0.0)
        h2 = jnp.dot(t, w_ref[...], preferred_element_type=jnp.float32)
        o_ref[...] = h2 * di_ref[...]

    return pl.pallas_call(
        body,
        grid=(n // _BM,),
        in_specs=[
            pl.BlockSpec((2, _BM, d), lambda i: (0, i, 0)),
            pl.BlockSpec((_BM, d), lambda i: (i, 0)),
            pl.BlockSpec((_BM, d), lambda i: (i, 0)),
            pl.BlockSpec((1, d), lambda i: (0, 0)),
            pl.BlockSpec((d, d), lambda i: (0, 0)),
        ],
        out_specs=pl.BlockSpec((_BM, d), lambda i: (i, 0)),
        out_shape=jax.ShapeDtypeStruct((n, d), jnp.float32),
    )(agg1, h1s, dinvb, b1, w2)


def _tc_tail(agg2, h2s, dinvb, b2, batch_r, fc1_w, fc1_b, fc2_w, fc2_b, n):
    d = h2s.shape[1]
    c = fc2_w.shape[1]
    nblk = n // _BM

    def body(a_ref, hs_ref, di_ref, b_ref, bat_ref, w1_ref, bb1_ref,
             w2_ref, bb2_ref, o_ref, pooled):
        i = pl.program_id(0)
        t = (a_ref[0] + a_ref[1] + hs_ref[...]) * di_ref[...] + b_ref[...]
        t = jnp.maximum(t, 0.0)
        bvec = bat_ref[0, 0, :]
        oh = (bvec[:, None] == lax.broadcasted_iota(jnp.int32, (_BM, _G), 1))
        oh = oh.astype(jnp.float32)
        contrib = lax.dot_general(oh, t, (((0,), (0,)), ((), ())),
                                  preferred_element_type=jnp.float32)

        @pl.when(i == 0)
        def _():
            pooled[...] = contrib

        @pl.when(i > 0)
        def _():
            pooled[...] += contrib

        @pl.when(i == nblk - 1)
        def _():
            p = pooled[...]
            f = jnp.dot(p, w1_ref[...], preferred_element_type=jnp.float32)
            f = jnp.maximum(f + bb1_ref[...], 0.0)
            logits = jnp.dot(f, w2_ref[...],
                             preferred_element_type=jnp.float32) + bb2_ref[...]
            m = jnp.max(logits, axis=-1, keepdims=True)
            lse = m + jnp.log(jnp.sum(jnp.exp(logits - m), axis=-1,
                                      keepdims=True))
            o_ref[...] = logits - lse

    return pl.pallas_call(
        body,
        grid=(nblk,),
        in_specs=[
            pl.BlockSpec((2, _BM, d), lambda i: (0, i, 0)),
            pl.BlockSpec((_BM, d), lambda i: (i, 0)),
            pl.BlockSpec((_BM, d), lambda i: (i, 0)),
            pl.BlockSpec((1, d), lambda i: (0, 0)),
            pl.BlockSpec((1, 1, _BM), lambda i: (i, 0, 0)),
            pl.BlockSpec((d, d), lambda i: (0, 0)),
            pl.BlockSpec((1, d), lambda i: (0, 0)),
            pl.BlockSpec((d, c), lambda i: (0, 0)),
            pl.BlockSpec((1, c), lambda i: (0, 0)),
        ],
        out_specs=pl.BlockSpec((_G, c), lambda i: (0, 0)),
        out_shape=jax.ShapeDtypeStruct((_G, c), jnp.float32),
        scratch_shapes=[pltpu.VMEM((_G, d), jnp.float32)],
    )(agg2, h2s, dinvb, b2, batch_r, fc1_w, fc1_b, fc2_w, fc2_b)


def kernel(x, edge_index, batch, W1, b1, W2, b2, fc1_W, fc1_b, fc2_W, fc2_b):
    n, d = x.shape
    e = edge_index.shape[1]
    nw = _NC * _NS

    n_ch = -(-e // (nw * _CH))
    if n_ch % 2:
        n_ch += 1
    e_pad = n_ch * _CH * nw
    np_rows = ((n + 1 + 127) // 128) * 128

    src_p = jnp.concatenate(
        [edge_index[0], jnp.full((e_pad - e,), n, jnp.int32)]
    ).reshape(nw * n_ch, _CH)
    dst_p = jnp.concatenate(
        [edge_index[1], jnp.full((e_pad - e,), n, jnp.int32)]
    ).reshape(nw * n_ch, _CH)
    ones16 = jnp.ones((_CH, 16), jnp.float32)
    zeros16 = jnp.zeros((np_rows, 16), jnp.float32)
    zeros_d = jnp.zeros((np_rows, d), jnp.float32)
    pad_rows = jnp.zeros((np_rows - n, d), jnp.float32)

    deg_p = _sc_degree(dst_p.reshape(-1), ones16, zeros16,
                       np_rows).reshape(_NC, np_rows, 16)
    h1 = _tc_matmul(x, W1)
    h1s, dinvb = _tc_scale(deg_p, h1, n)

    agg1 = _sc_aggregate(jnp.concatenate([h1s, pad_rows]), src_p, dst_p,
                         zeros_d, np_rows).reshape(_NC, np_rows, d)
    h2s = _tc_mid(agg1, h1s, dinvb, b1.reshape(1, -1), W2, n)

    agg2 = _sc_aggregate(jnp.concatenate([h2s, pad_rows]), src_p, dst_p,
                         zeros_d, np_rows).reshape(_NC, np_rows, d)
    nblk = n // _BM
    out = _tc_tail(agg2, h2s, dinvb, b2.reshape(1, -1),
                   batch.reshape(nblk, 1, _BM), fc1_W, fc1_b.reshape(1, -1),
                   fc2_W, fc2_b.reshape(1, -1), n)
    return out
